# baseline (speedup 1.0000x reference)
import functools

import jax
import jax.numpy as jnp
from jax.experimental import pallas as pl
from jax.experimental.pallas import tpu as pltpu

N = 500000
NUM_VOX = 50000
BN = 1000
BV = 1000


def _dot_t(x, w):
    return jax.lax.dot_general(x, w, (((1,), (1,)), ((), ())),
                               preferred_element_type=jnp.float32)


def _stage1_body(x_ref, rgb_ref, w1_ref, b1_ref, wr_ref, br_ref,
                 w2a_ref, w2b_ref, b2_ref, out_ref):
    a = jnp.maximum(_dot_t(x_ref[...], w1_ref[...]) + b1_ref[...], 0.0)
    b = jnp.maximum(_dot_t(rgb_ref[...], wr_ref[...]) + br_ref[...], 0.0)
    h = _dot_t(a, w2a_ref[...]) + _dot_t(b, w2b_ref[...]) + b2_ref[...]
    out_ref[...] = jnp.maximum(h, 0.0)


def _stage2_body(pgf_ref, pf2_ref, w3a_ref, w3b_ref, b3_ref,
                 w4_ref, b4_ref, out_ref):
    h = (_dot_t(pgf_ref[...], w3a_ref[...]) + _dot_t(pf2_ref[...], w3b_ref[...])
         + b3_ref[...])
    pf4 = jnp.maximum(h, 0.0)
    out_ref[...] = jnp.maximum(_dot_t(pf4, w4_ref[...]) + b4_ref[...], 0.0)


def _mlp_body(x_ref, w_ref, b_ref, out_ref):
    out_ref[...] = jnp.maximum(_dot_t(x_ref[...], w_ref[...]) + b_ref[...], 0.0)


def _full(shape):
    return pl.BlockSpec(shape, lambda i: (0,) * len(shape))


def _rows(bn, d):
    return pl.BlockSpec((bn, d), lambda i: (i, 0))


def _stage1(x, rgb, w1, b1, wr, br, w2, b2):
    x8 = jnp.pad(x, ((0, 0), (0, 2)))
    w18 = jnp.pad(w1, ((0, 0), (0, 2)))
    w2a, w2b = w2[:, :64], w2[:, 64:]
    grid = N // BN
    return pl.pallas_call(
        _stage1_body,
        grid=(grid,),
        in_specs=[_rows(BN, 8), _rows(BN, 128), _full((64, 8)), _full((1, 64)),
                  _full((64, 128)), _full((1, 64)), _full((128, 64)),
                  _full((128, 64)), _full((1, 128))],
        out_specs=_rows(BN, 128),
        out_shape=jax.ShapeDtypeStruct((N, 128), jnp.float32),
        compiler_params=pltpu.CompilerParams(
            dimension_semantics=("arbitrary",)),
    )(x8, rgb, w18, b1[None, :], wr, br[None, :], w2a, w2b, b2[None, :])


def _stage2(pgf, pf2, w3, b3, w4, b4):
    w3a, w3b = w3[:, :128], w3[:, 128:]
    grid = N // BN
    return pl.pallas_call(
        _stage2_body,
        grid=(grid,),
        in_specs=[_rows(BN, 128), _rows(BN, 128), _full((256, 128)),
                  _full((256, 128)), _full((1, 256)), _full((256, 256)),
                  _full((1, 256))],
        out_specs=_rows(BN, 256),
        out_shape=jax.ShapeDtypeStruct((N, 256), jnp.float32),
        compiler_params=pltpu.CompilerParams(
            dimension_semantics=("arbitrary",)),
    )(pgf, pf2, w3a, w3b, b3[None, :], w4, b4[None, :])


def _vox_mlp(x, w, b):
    v, d = x.shape
    grid = v // BV
    return pl.pallas_call(
        _mlp_body,
        grid=(grid,),
        in_specs=[_rows(BV, d), _full(w.shape), _full((1, d))],
        out_specs=_rows(BV, d),
        out_shape=jax.ShapeDtypeStruct((v, d), jnp.float32),
        compiler_params=pltpu.CompilerParams(
            dimension_semantics=("arbitrary",)),
    )(x, w, b[None, :])


def kernel(inp_feat, vox2point_idx, point_rgb_feat, W1, b1, Wr, br, W2, b2,
           Wv1, bv1, W3, b3, W4, b4, Wv2, bv2):
    pf2 = _stage1(inp_feat, point_rgb_feat, W1, b1, Wr, br, W2, b2)
    vox1 = jax.ops.segment_max(pf2, vox2point_idx, num_segments=NUM_VOX)
    vox1 = jnp.where(jnp.isfinite(vox1), vox1, 0.0)
    occ = _vox_mlp(vox1, Wv1, bv1)
    pgf = occ[vox2point_idx]
    pf5 = _stage2(pgf, pf2, W3, b3, W4, b4)
    vox2 = jax.ops.segment_max(pf5, vox2point_idx, num_segments=NUM_VOX)
    vox2 = jnp.where(jnp.isfinite(vox2), vox2, 0.0)
    return _vox_mlp(vox2, Wv2, bv2)
